# baseline (speedup 1.0000x reference)
import jax
import jax.numpy as jnp
from jax.experimental import pallas as pl
from jax.experimental.pallas import tpu as pltpu

N_RNA = 1024
N_DIS = 512
HIDDEN = 128
NSTEP = 4
CHROW = N_RNA // NSTEP


def _mm(a, b):
    return jax.lax.dot_general(a, b, (((1,), (0,)), ((), ())),
                               preferred_element_type=jnp.float32)


def _mmT(a, b):
    return jax.lax.dot_general(a, b, (((0,), (0,)), ((), ())),
                               preferred_element_type=jnp.float32)


def _bf(v):
    return v.astype(jnp.bfloat16)


def _inv_deg(deg):
    return jnp.where(deg > 0, 1.0 / jnp.where(deg > 0, deg, 1.0), 0.0)


def _fused(mf_h, dss_h, w1d_h, dgs_h, wld_h, df_h,
           cf, cgs, W1m, Wlm, W2m, W2d, b1m, b2m, b1d, b2d, blm, bld,
           out_ref,
           mf_v, dss_v, w1d_v, dgs_v, wld_v, df_v, xw_s, o2_s,
           s0, s1, s2, s3, s4, s5):
    pid = pl.program_id(0)
    cp_mf = pltpu.make_async_copy(mf_h, mf_v, s0)
    cp_dss = pltpu.make_async_copy(dss_h, dss_v, s1)
    cp_w1d = pltpu.make_async_copy(w1d_h, w1d_v, s2)
    cp_dgs = pltpu.make_async_copy(dgs_h, dgs_v, s3)
    cp_wld = pltpu.make_async_copy(wld_h, wld_v, s4)
    cp_df = pltpu.make_async_copy(df_h, df_v, s5)

    @pl.when(pid == 0)
    def _():
        for c in (cp_mf, cp_dss, cp_w1d, cp_dgs, cp_wld, cp_df):
            c.start()

    w1b = _bf(W1m[...])
    wlb = _bf(Wlm[...])
    x1b = _bf(cf[...])
    x2b = _bf(cgs[...])
    row = pid * CHROW
    xw_s[pl.ds(row, CHROW), :] = (
        _mm(x1b, w1b[:N_RNA]) + _mm(x2b, w1b[N_RNA:]))
    o2_s[pl.ds(row, CHROW), :] = (
        _mm(x1b, wlb[:N_RNA]) + _mm(x2b, wlb[N_RNA:]))

    @pl.when(pid == NSTEP - 1)
    def _():
        cp_mf.wait()
        adjf = mf_v[...]
        Binv = _inv_deg(jnp.sum(adjf, axis=1, keepdims=True,
                                dtype=jnp.float32))
        Dinv = _inv_deg(jnp.sum(adjf, axis=0, keepdims=True,
                                dtype=jnp.float32)).T
        adj = _bf(adjf)
        e1 = _mm(adj, _bf(xw_s[...])) * Binv
        cp_dss.wait()
        cp_w1d.wait()
        w1d_b = _bf(w1d_v[...])
        y1b = _bf(dss_v[...])
        xwd = _mm(y1b, w1d_b[:N_DIS])
        h = jnp.maximum(_mmT(adj, _bf(e1)) * Dinv + b1m[...], 0.0)
        cp_dgs.wait()
        y2b = _bf(dgs_v[...])
        xwd = xwd + _mm(y2b, w1d_b[N_DIS:])
        xw2 = _mm(_bf(h), _bf(W2m[...]))
        cp_wld.wait()
        wld_b = _bf(wld_v[...])
        o2d = _mm(y1b, wld_b[:N_DIS]) + _mm(y2b, wld_b[N_DIS:]) + bld[...]
        e2 = _mm(adj, _bf(xw2)) * Binv
        cp_df.wait()
        dff = df_v[...]
        Binv_d = _inv_deg(jnp.sum(dff, axis=1, keepdims=True,
                                  dtype=jnp.float32))
        Dinv_d = _inv_deg(jnp.sum(dff, axis=0, keepdims=True,
                                  dtype=jnp.float32)).T
        adj_d = _bf(dff)
        o1m = _mmT(adj, _bf(e2)) * Dinv + b2m[...]
        out_ref[:N_RNA, :] = (o1m + o2_s[...] + blm[...]) * 0.5
        e1d = _mm(adj_d, _bf(xwd)) * Binv_d
        hd = jnp.maximum(_mmT(adj_d, _bf(e1d)) * Dinv_d + b1d[...], 0.0)
        e2d = _mm(adj_d, _bf(_mm(_bf(hd), _bf(W2d[...])))) * Binv_d
        o1d = _mmT(adj_d, _bf(e2d)) * Dinv_d + b2d[...]
        out_ref[N_RNA:, :] = (o1d + o2d) * 0.5


def kernel(m_f, d_f, c_func, c_gs, d_ss, d_gs, W1m, b1m, W2m, b2m,
           W1d, b1d, W2d, b2d, Wlm, blm, Wld, bld):
    f32 = jnp.float32
    hbm_spec = pl.BlockSpec(memory_space=pltpu.MemorySpace.HBM)
    chunk_spec = pl.BlockSpec((CHROW, N_RNA), lambda i: (i, 0))
    full = pl.BlockSpec((N_RNA + N_DIS, HIDDEN), lambda i: (0, 0))
    w_spec = lambda r: pl.BlockSpec((r, HIDDEN), lambda i: (0, 0))
    call = pl.pallas_call(
        _fused,
        grid=(NSTEP,),
        out_shape=jax.ShapeDtypeStruct((N_RNA + N_DIS, HIDDEN), f32),
        in_specs=[hbm_spec] * 6 + [chunk_spec, chunk_spec]
        + [w_spec(2 * N_RNA), w_spec(2 * N_RNA), w_spec(HIDDEN),
           w_spec(HIDDEN)] + [w_spec(1)] * 6,
        out_specs=full,
        scratch_shapes=[
            pltpu.VMEM((N_RNA, N_RNA), f32),
            pltpu.VMEM((N_DIS, N_DIS), f32),
            pltpu.VMEM((2 * N_DIS, HIDDEN), f32),
            pltpu.VMEM((N_DIS, N_DIS), f32),
            pltpu.VMEM((2 * N_DIS, HIDDEN), f32),
            pltpu.VMEM((N_DIS, N_DIS), f32),
            pltpu.VMEM((N_RNA, HIDDEN), f32),
            pltpu.VMEM((N_RNA, HIDDEN), f32),
        ] + [pltpu.SemaphoreType.DMA] * 6,
        compiler_params=pltpu.CompilerParams(
            dimension_semantics=("arbitrary",)),
    )
    return call(
        m_f, d_ss, W1d, d_gs, Wld, d_f,
        c_func, c_gs, W1m, Wlm, W2m, W2d,
        b1m.reshape(1, HIDDEN), b2m.reshape(1, HIDDEN),
        b1d.reshape(1, HIDDEN), b2d.reshape(1, HIDDEN),
        blm.reshape(1, HIDDEN), bld.reshape(1, HIDDEN))


# speedup vs baseline: 1.1951x; 1.1951x over previous
import jax
import jax.numpy as jnp
from jax.experimental import pallas as pl
from jax.experimental.pallas import tpu as pltpu

N_RNA = 1024
N_DIS = 512
HIDDEN = 128


def _mm(a, b):
    return jax.lax.dot_general(a, b, (((1,), (0,)), ((), ())),
                               preferred_element_type=jnp.float32)


def _mmT(a, b):
    return jax.lax.dot_general(a, b, (((0,), (0,)), ((), ())),
                               preferred_element_type=jnp.float32)


def _bf(v):
    return v.astype(jnp.bfloat16)


def _inv_deg(deg):
    return jnp.where(deg > 0, 1.0 / jnp.where(deg > 0, deg, 1.0), 0.0)


def _conv_chain(adj_f32, xw, b1, W2, b2):
    Bd = jnp.sum(adj_f32, axis=1, keepdims=True, dtype=jnp.float32)
    Dd = jnp.sum(adj_f32, axis=0, keepdims=True, dtype=jnp.float32).T
    Binv = _inv_deg(Bd)
    Dinv = _inv_deg(Dd)
    adj = _bf(adj_f32)
    e1 = _mm(adj, _bf(xw)) * Binv
    h = jnp.maximum(_mmT(adj, _bf(e1)) * Dinv + b1[...], 0.0)
    e2 = _mm(adj, _bf(_mm(_bf(h), _bf(W2[...])))) * Binv
    return _mmT(adj, _bf(e2)) * Dinv + b2[...]


def _fused(mf_h, dss_h, w1d_h, dgs_h, wld_h, df_h,
           cf, cgs, W1m, Wlm, W2m, W2d, b1m, b2m, b1d, b2d, blm, bld,
           out_ref,
           mf_v, dss_v, w1d_v, dgs_v, wld_v, df_v, xw_s, o2_s,
           s0, s1, s2, s3, s4, s5):
    pid = pl.program_id(0)
    cp_mf = pltpu.make_async_copy(mf_h, mf_v, s0)
    cp_dss = pltpu.make_async_copy(dss_h, dss_v, s1)
    cp_w1d = pltpu.make_async_copy(w1d_h, w1d_v, s2)
    cp_dgs = pltpu.make_async_copy(dgs_h, dgs_v, s3)
    cp_wld = pltpu.make_async_copy(wld_h, wld_v, s4)
    cp_df = pltpu.make_async_copy(df_h, df_v, s5)

    @pl.when(pid == 0)
    def _():
        for c in (cp_mf, cp_dss, cp_w1d, cp_dgs, cp_wld, cp_df):
            c.start()

    w1b = _bf(W1m[...])
    wlb = _bf(Wlm[...])
    x1b = _bf(cf[...])
    x2b = _bf(cgs[...])
    half = pid * (N_RNA // 2)
    xw_s[pl.ds(half, N_RNA // 2), :] = (
        _mm(x1b, w1b[:N_RNA]) + _mm(x2b, w1b[N_RNA:]))
    o2_s[pl.ds(half, N_RNA // 2), :] = (
        _mm(x1b, wlb[:N_RNA]) + _mm(x2b, wlb[N_RNA:]))

    @pl.when(pid == 1)
    def _():
        cp_mf.wait()
        adjf = mf_v[...]
        Binv = _inv_deg(jnp.sum(adjf, axis=1, keepdims=True,
                                dtype=jnp.float32))
        Dinv = _inv_deg(jnp.sum(adjf, axis=0, keepdims=True,
                                dtype=jnp.float32)).T
        adj = _bf(adjf)
        e1 = _mm(adj, _bf(xw_s[...])) * Binv
        cp_dss.wait()
        cp_w1d.wait()
        w1d_b = _bf(w1d_v[...])
        y1b = _bf(dss_v[...])
        xwd = _mm(y1b, w1d_b[:N_DIS])
        h = jnp.maximum(_mmT(adj, _bf(e1)) * Dinv + b1m[...], 0.0)
        cp_dgs.wait()
        y2b = _bf(dgs_v[...])
        xwd = xwd + _mm(y2b, w1d_b[N_DIS:])
        xw2 = _mm(_bf(h), _bf(W2m[...]))
        cp_wld.wait()
        wld_b = _bf(wld_v[...])
        o2d = _mm(y1b, wld_b[:N_DIS]) + _mm(y2b, wld_b[N_DIS:]) + bld[...]
        e2 = _mm(adj, _bf(xw2)) * Binv
        cp_df.wait()
        dff = df_v[...]
        Binv_d = _inv_deg(jnp.sum(dff, axis=1, keepdims=True,
                                  dtype=jnp.float32))
        Dinv_d = _inv_deg(jnp.sum(dff, axis=0, keepdims=True,
                                  dtype=jnp.float32)).T
        adj_d = _bf(dff)
        o1m = _mmT(adj, _bf(e2)) * Dinv + b2m[...]
        out_ref[:N_RNA, :] = (o1m + o2_s[...] + blm[...]) * 0.5
        e1d = _mm(adj_d, _bf(xwd)) * Binv_d
        hd = jnp.maximum(_mmT(adj_d, _bf(e1d)) * Dinv_d + b1d[...], 0.0)
        e2d = _mm(adj_d, _bf(_mm(_bf(hd), _bf(W2d[...])))) * Binv_d
        o1d = _mmT(adj_d, _bf(e2d)) * Dinv_d + b2d[...]
        out_ref[N_RNA:, :] = (o1d + o2d) * 0.5


def kernel(m_f, d_f, c_func, c_gs, d_ss, d_gs, W1m, b1m, W2m, b2m,
           W1d, b1d, W2d, b2d, Wlm, blm, Wld, bld):
    f32 = jnp.float32
    hbm_spec = pl.BlockSpec(memory_space=pltpu.MemorySpace.HBM)
    vmem_spec = pl.BlockSpec(memory_space=pltpu.MemorySpace.VMEM)
    half_spec = pl.BlockSpec((N_RNA // 2, N_RNA), lambda i: (i, 0))
    full = pl.BlockSpec((N_RNA + N_DIS, HIDDEN), lambda i: (0, 0))
    w_spec = lambda r: pl.BlockSpec((r, HIDDEN), lambda i: (0, 0))
    call = pl.pallas_call(
        _fused,
        grid=(2,),
        out_shape=jax.ShapeDtypeStruct((N_RNA + N_DIS, HIDDEN), f32),
        in_specs=[hbm_spec] * 6 + [half_spec, half_spec]
        + [w_spec(2 * N_RNA), w_spec(2 * N_RNA), w_spec(HIDDEN),
           w_spec(HIDDEN)] + [w_spec(1)] * 6,
        out_specs=full,
        scratch_shapes=[
            pltpu.VMEM((N_RNA, N_RNA), f32),
            pltpu.VMEM((N_DIS, N_DIS), f32),
            pltpu.VMEM((2 * N_DIS, HIDDEN), f32),
            pltpu.VMEM((N_DIS, N_DIS), f32),
            pltpu.VMEM((2 * N_DIS, HIDDEN), f32),
            pltpu.VMEM((N_DIS, N_DIS), f32),
            pltpu.VMEM((N_RNA, HIDDEN), f32),
            pltpu.VMEM((N_RNA, HIDDEN), f32),
        ] + [pltpu.SemaphoreType.DMA] * 6,
        compiler_params=pltpu.CompilerParams(
            dimension_semantics=("arbitrary",)),
    )
    return call(
        m_f, d_ss, W1d, d_gs, Wld, d_f,
        c_func, c_gs, W1m, Wlm, W2m, W2d,
        b1m.reshape(1, HIDDEN), b2m.reshape(1, HIDDEN),
        b1d.reshape(1, HIDDEN), b2d.reshape(1, HIDDEN),
        blm.reshape(1, HIDDEN), bld.reshape(1, HIDDEN))
